# Initial kernel scaffold; baseline (speedup 1.0000x reference)
import jax
import jax.numpy as jnp
from jax.experimental import pallas as pl


def kernel(x, feature_grid_3d, plane0, plane1, plane2):
    raise NotImplementedError("write your pallas kernel here")


# speedup vs baseline: 1.5096x; 1.5096x over previous
import jax
import jax.numpy as jnp
from jax import lax
from jax.experimental import pallas as pl
from jax.experimental.pallas import tpu as pltpu
from jax.experimental.pallas import tpu_sc as plsc

_NC, _NS, _L = 2, 16, 16
_NW = _NC * _NS
_P = 128
_PLANE_DIMID = ((0, 1), (0, 2), (1, 2))
_NT = 20


def _make_sc_kernel(B, C, vol_dims, pdims, interpret=False):
    D, Hg, Wg = vol_dims
    ppt = B // _NW
    P = min(_P, ppt)
    nchunks = ppt // P
    ngroups = P // _L

    mesh = plsc.VectorSubcoreMesh(core_axis_name="c", subcore_axis_name="s",
                                  num_cores=_NC, num_subcores=_NS)
    out_type = jax.ShapeDtypeStruct((C, B), jnp.float32)
    scratch = [
        pltpu.VMEM((P * 6,), jnp.float32),
        pltpu.VMEM((_NT, P), jnp.int32),
        pltpu.VMEM((_NT, P), jnp.float32),
        pltpu.VMEM((_NT, P, C), jnp.float32),
        pltpu.VMEM((C, P), jnp.float32),
        pltpu.SemaphoreType.DMA,
    ]

    def body(xf, g3, p0, p1, p2, out, xv, idx_r, w_r, rows_r, outv, sem):
        wid = lax.axis_index("s") * _NC + lax.axis_index("c")
        tile_base = wid * ppt
        lane = lax.iota(jnp.int32, _L)

        def prep_group(g, c):
            sl = pl.ds(g * _L, _L)
            p6 = (lane + g * _L) * 6
            gx = plsc.load_gather(xv, [p6])
            gy = plsc.load_gather(xv, [p6 + 1])
            gz = plsc.load_gather(xv, [p6 + 2])
            coords = (gx, gy, gz)
            ix = (gx + 1.0) * 0.5 * (Wg - 1)
            iy = (gy + 1.0) * 0.5 * (Hg - 1)
            iz = (gz + 1.0) * 0.5 * (D - 1)
            xi = jnp.clip(ix.astype(jnp.int32), 0, Wg - 1)
            yi = jnp.clip(iy.astype(jnp.int32), 0, Hg - 1)
            zi = jnp.clip(iz.astype(jnp.int32), 0, D - 1)
            fx = ix - xi.astype(jnp.float32)
            fy = iy - yi.astype(jnp.float32)
            fz = iz - zi.astype(jnp.float32)
            dx = jnp.minimum(xi + 1, Wg - 1) - xi
            dy = (jnp.minimum(yi + 1, Hg - 1) - yi) * Wg
            dz = (jnp.minimum(zi + 1, D - 1) - zi) * (Wg * Hg)
            base3 = (zi * Hg + yi) * Wg + xi
            idx_r[0, sl] = base3
            idx_r[1, sl] = base3 + dx
            idx_r[2, sl] = base3 + dy
            idx_r[3, sl] = base3 + dy + dx
            idx_r[4, sl] = base3 + dz
            idx_r[5, sl] = base3 + dz + dx
            idx_r[6, sl] = base3 + dz + dy
            idx_r[7, sl] = base3 + dz + dy + dx
            ox = 1.0 - fx
            oy = 1.0 - fy
            oz = 1.0 - fz
            w_r[0, sl] = ox * oy * oz
            w_r[1, sl] = fx * oy * oz
            w_r[2, sl] = ox * fy * oz
            w_r[3, sl] = fx * fy * oz
            w_r[4, sl] = ox * oy * fz
            w_r[5, sl] = fx * oy * fz
            w_r[6, sl] = ox * fy * fz
            w_r[7, sl] = fx * fy * fz
            for k, (a, b) in enumerate(_PLANE_DIMID):
                PHk, PWk = pdims[k]
                gu = coords[a]
                gv = coords[b]
                iu = (gu + 1.0) * 0.5 * (PWk - 1)
                iv = (gv + 1.0) * 0.5 * (PHk - 1)
                ui = jnp.clip(iu.astype(jnp.int32), 0, PWk - 1)
                vi = jnp.clip(iv.astype(jnp.int32), 0, PHk - 1)
                fu = iu - ui.astype(jnp.float32)
                fv = iv - vi.astype(jnp.float32)
                du = jnp.minimum(ui + 1, PWk - 1) - ui
                dv = (jnp.minimum(vi + 1, PHk - 1) - vi) * PWk
                r = vi * PWk + ui
                t0 = 8 + 4 * k
                idx_r[t0 + 0, sl] = r
                idx_r[t0 + 1, sl] = r + du
                idx_r[t0 + 2, sl] = r + dv
                idx_r[t0 + 3, sl] = r + dv + du
                ou = 1.0 - fu
                ov = 1.0 - fv
                w_r[t0 + 0, sl] = ou * ov
                w_r[t0 + 1, sl] = fu * ov
                w_r[t0 + 2, sl] = ou * fv
                w_r[t0 + 3, sl] = fu * fv
            return c

        def combine_group(g, c):
            sl = pl.ds(g * _L, _L)
            pvec = lane + g * _L
            wvs = [w_r[t, sl] for t in range(_NT)]
            for f in range(C):
                fvec = jnp.full((_L,), f, jnp.int32)
                a = None
                for t in range(8):
                    tv = jnp.full((_L,), t, jnp.int32)
                    v = plsc.load_gather(rows_r, [tv, pvec, fvec])
                    a = wvs[t] * v if a is None else a + wvs[t] * v
                for k in range(3):
                    m = None
                    for t in range(8 + 4 * k, 12 + 4 * k):
                        tv = jnp.full((_L,), t, jnp.int32)
                        v = plsc.load_gather(rows_r, [tv, pvec, fvec])
                        m = wvs[t] * v if m is None else m + wvs[t] * v
                    a = a * m
                outv[f, sl] = a
            return c

        tabs = (g3,) * 8 + (p0,) * 4 + (p1,) * 4 + (p2,) * 4

        def chunk(kk, c):
            base = tile_base + kk * P
            pltpu.sync_copy(xf.at[pl.ds(base * 6, P * 6)], xv)
            lax.fori_loop(0, ngroups, prep_group, 0)
            descs = [pltpu.async_copy(tabs[t].at[idx_r.at[t]], rows_r.at[t], sem)
                     for t in range(_NT)]
            for d in descs:
                d.wait()
            lax.fori_loop(0, ngroups, combine_group, 0)
            pltpu.sync_copy(outv, out.at[:, pl.ds(base, P)])
            return c

        lax.fori_loop(0, nchunks, chunk, 0)

    return pl.kernel(body, out_type=out_type, mesh=mesh,
                     scratch_types=scratch, interpret=interpret,
                     compiler_params=pltpu.CompilerParams(
                         needs_layout_passes=False,
                         use_tc_tiling_on_sc=False))


def kernel(x, feature_grid_3d, plane0, plane1, plane2):
    B = x.shape[0]
    C = feature_grid_3d.shape[1]
    D, Hg, Wg = feature_grid_3d.shape[2:5]
    assert C == _L and B % (_NW * _L) == 0
    g3t = jnp.transpose(feature_grid_3d[0], (1, 2, 3, 0)).reshape(D * Hg * Wg, C)
    pts, pdims = [], []
    for p in (plane0, plane1, plane2):
        ph, pw = p.shape[2], p.shape[3]
        pdims.append((ph, pw))
        pts.append(jnp.transpose(p[0], (1, 2, 0)).reshape(ph * pw, C))
    k = _make_sc_kernel(B, C, (D, Hg, Wg), tuple(pdims))
    return k(x.reshape(-1), g3t, pts[0], pts[1], pts[2])


# speedup vs baseline: 1.6437x; 1.0888x over previous
import jax
import jax.numpy as jnp
from jax import lax
from jax.experimental import pallas as pl
from jax.experimental.pallas import tpu as pltpu
from jax.experimental.pallas import tpu_sc as plsc

_NC, _NS, _L = 2, 16, 16
_NW = _NC * _NS
_P = 128
_PLANE_DIMID = ((0, 1), (0, 2), (1, 2))
_NT = 20


def _make_sc_kernel(B, C, vol_dims, pdims, interpret=False):
    D, Hg, Wg = vol_dims
    ppt = B // _NW
    P = min(_P, ppt)
    nchunks = ppt // P
    ngroups = P // _L
    pipelined = nchunks % 2 == 0 and nchunks >= 2

    mesh = plsc.VectorSubcoreMesh(core_axis_name="c", subcore_axis_name="s",
                                  num_cores=_NC, num_subcores=_NS)
    out_type = jax.ShapeDtypeStruct((C, B), jnp.float32)
    scratch = [
        pltpu.VMEM((P * 6,), jnp.float32),
        pltpu.VMEM((_NT, P), jnp.int32),
        pltpu.VMEM((_NT, P), jnp.int32),
        pltpu.VMEM((_NT, P), jnp.float32),
        pltpu.VMEM((_NT, P), jnp.float32),
        pltpu.VMEM((_NT, P, C), jnp.float32),
        pltpu.VMEM((_NT, P, C), jnp.float32),
        pltpu.VMEM((C, P), jnp.float32),
        pltpu.SemaphoreType.DMA,
        pltpu.SemaphoreType.DMA,
    ]

    def body(xf, g3, p0, p1, p2, out, xv, idx0, idx1, w0, w1, rows0, rows1,
             outv, sem0, sem1):
        wid = lax.axis_index("s") * _NC + lax.axis_index("c")
        tile_base = wid * ppt
        lane = lax.iota(jnp.int32, _L)
        tabs = (g3,) * 8 + (p0,) * 4 + (p1,) * 4 + (p2,) * 4
        idxs = (idx0, idx1)
        ws = (w0, w1)
        rows = (rows0, rows1)
        sems = (sem0, sem1)

        def make_prep(idx_r, w_r):
            def prep_group(g, c):
                sl = pl.ds(g * _L, _L)
                p6 = (lane + g * _L) * 6
                gx = plsc.load_gather(xv, [p6])
                gy = plsc.load_gather(xv, [p6 + 1])
                gz = plsc.load_gather(xv, [p6 + 2])
                coords = (gx, gy, gz)
                ix = (gx + 1.0) * 0.5 * (Wg - 1)
                iy = (gy + 1.0) * 0.5 * (Hg - 1)
                iz = (gz + 1.0) * 0.5 * (D - 1)
                xi = jnp.clip(ix.astype(jnp.int32), 0, Wg - 1)
                yi = jnp.clip(iy.astype(jnp.int32), 0, Hg - 1)
                zi = jnp.clip(iz.astype(jnp.int32), 0, D - 1)
                fx = ix - xi.astype(jnp.float32)
                fy = iy - yi.astype(jnp.float32)
                fz = iz - zi.astype(jnp.float32)
                dx = jnp.minimum(xi + 1, Wg - 1) - xi
                dy = (jnp.minimum(yi + 1, Hg - 1) - yi) * Wg
                dz = (jnp.minimum(zi + 1, D - 1) - zi) * (Wg * Hg)
                base3 = (zi * Hg + yi) * Wg + xi
                idx_r[0, sl] = base3
                idx_r[1, sl] = base3 + dx
                idx_r[2, sl] = base3 + dy
                idx_r[3, sl] = base3 + dy + dx
                idx_r[4, sl] = base3 + dz
                idx_r[5, sl] = base3 + dz + dx
                idx_r[6, sl] = base3 + dz + dy
                idx_r[7, sl] = base3 + dz + dy + dx
                ox = 1.0 - fx
                oy = 1.0 - fy
                oz = 1.0 - fz
                w_r[0, sl] = ox * oy * oz
                w_r[1, sl] = fx * oy * oz
                w_r[2, sl] = ox * fy * oz
                w_r[3, sl] = fx * fy * oz
                w_r[4, sl] = ox * oy * fz
                w_r[5, sl] = fx * oy * fz
                w_r[6, sl] = ox * fy * fz
                w_r[7, sl] = fx * fy * fz
                for k, (a, b) in enumerate(_PLANE_DIMID):
                    PHk, PWk = pdims[k]
                    gu = coords[a]
                    gv = coords[b]
                    iu = (gu + 1.0) * 0.5 * (PWk - 1)
                    iv = (gv + 1.0) * 0.5 * (PHk - 1)
                    ui = jnp.clip(iu.astype(jnp.int32), 0, PWk - 1)
                    vi = jnp.clip(iv.astype(jnp.int32), 0, PHk - 1)
                    fu = iu - ui.astype(jnp.float32)
                    fv = iv - vi.astype(jnp.float32)
                    du = jnp.minimum(ui + 1, PWk - 1) - ui
                    dv = (jnp.minimum(vi + 1, PHk - 1) - vi) * PWk
                    r = vi * PWk + ui
                    t0 = 8 + 4 * k
                    idx_r[t0 + 0, sl] = r
                    idx_r[t0 + 1, sl] = r + du
                    idx_r[t0 + 2, sl] = r + dv
                    idx_r[t0 + 3, sl] = r + dv + du
                    ou = 1.0 - fu
                    ov = 1.0 - fv
                    w_r[t0 + 0, sl] = ou * ov
                    w_r[t0 + 1, sl] = fu * ov
                    w_r[t0 + 2, sl] = ou * fv
                    w_r[t0 + 3, sl] = fu * fv
                return c
            return prep_group

        def make_combine(rows_r, w_r):
            def combine_group(g, c):
                sl = pl.ds(g * _L, _L)
                pvec = lane + g * _L
                wvs = [w_r[t, sl] for t in range(_NT)]
                for f in range(C):
                    fvec = jnp.full((_L,), f, jnp.int32)
                    a = None
                    for t in range(8):
                        tv = jnp.full((_L,), t, jnp.int32)
                        v = plsc.load_gather(rows_r, [tv, pvec, fvec])
                        a = wvs[t] * v if a is None else a + wvs[t] * v
                    for k in range(3):
                        m = None
                        for t in range(8 + 4 * k, 12 + 4 * k):
                            tv = jnp.full((_L,), t, jnp.int32)
                            v = plsc.load_gather(rows_r, [tv, pvec, fvec])
                            m = wvs[t] * v if m is None else m + wvs[t] * v
                        a = a * m
                    outv[f, sl] = a
                return c
            return combine_group

        preps = (make_prep(idx0, w0), make_prep(idx1, w1))
        combines = (make_combine(rows0, w0), make_combine(rows1, w1))

        def stage_prep_fire(cc, par):
            base = tile_base + cc * P
            pltpu.sync_copy(xf.at[pl.ds(base * 6, P * 6)], xv)
            lax.fori_loop(0, ngroups, preps[par], 0)
            for t in range(_NT):
                pltpu.async_copy(tabs[t].at[idxs[par].at[t]], rows[par].at[t],
                                 sems[par])

        def wait_combine_store(cc, par):
            for t in range(_NT):
                pltpu.make_async_copy(tabs[t].at[idxs[par].at[t]],
                                      rows[par].at[t], sems[par]).wait()
            lax.fori_loop(0, ngroups, combines[par], 0)
            base = tile_base + cc * P
            pltpu.sync_copy(outv, out.at[:, pl.ds(base, P)])

        if pipelined:
            stage_prep_fire(0, 0)

            def pair(j, c):
                c0 = 2 * j
                stage_prep_fire(c0 + 1, 1)
                wait_combine_store(c0, 0)

                @pl.when(c0 + 2 < nchunks)
                def _fire_next():
                    stage_prep_fire(c0 + 2, 0)

                wait_combine_store(c0 + 1, 1)
                return c

            lax.fori_loop(0, nchunks // 2, pair, 0)
        else:
            def chunk(kk, c):
                stage_prep_fire(kk, 0)
                wait_combine_store(kk, 0)
                return c

            lax.fori_loop(0, nchunks, chunk, 0)

    return pl.kernel(body, out_type=out_type, mesh=mesh,
                     scratch_types=scratch, interpret=interpret,
                     compiler_params=pltpu.CompilerParams(
                         needs_layout_passes=False,
                         use_tc_tiling_on_sc=False))


def kernel(x, feature_grid_3d, plane0, plane1, plane2):
    B = x.shape[0]
    C = feature_grid_3d.shape[1]
    D, Hg, Wg = feature_grid_3d.shape[2:5]
    assert C == _L and B % (_NW * _L) == 0
    g3t = jnp.transpose(feature_grid_3d[0], (1, 2, 3, 0)).reshape(D * Hg * Wg, C)
    pts, pdims = [], []
    for p in (plane0, plane1, plane2):
        ph, pw = p.shape[2], p.shape[3]
        pdims.append((ph, pw))
        pts.append(jnp.transpose(p[0], (1, 2, 0)).reshape(ph * pw, C))
    k = _make_sc_kernel(B, C, (D, Hg, Wg), tuple(pdims))
    return k(x.reshape(-1), g3t, pts[0], pts[1], pts[2])


# speedup vs baseline: 2.1927x; 1.3340x over previous
import jax
import jax.numpy as jnp
from jax import lax
from jax.experimental import pallas as pl
from jax.experimental.pallas import tpu as pltpu
from jax.experimental.pallas import tpu_sc as plsc

_NC, _NS, _L = 2, 16, 16
_NW = _NC * _NS
_P = 128
_PLANE_DIMID = ((0, 1), (0, 2), (1, 2))
_NT = 20


def _make_sc_kernel(B, C, vol_dims, pdims, interpret=False):
    D, Hg, Wg = vol_dims
    ppt = B // _NW
    P = min(_P, ppt)
    nchunks = ppt // P
    ngroups = P // _L
    pipelined = nchunks % 2 == 0 and nchunks >= 2

    mesh = plsc.VectorSubcoreMesh(core_axis_name="c", subcore_axis_name="s",
                                  num_cores=_NC, num_subcores=_NS)
    out_type = jax.ShapeDtypeStruct((C, B), jnp.float32)
    scratch = [
        pltpu.VMEM((P * 6,), jnp.float32),
        pltpu.VMEM((_NT, P), jnp.int32),
        pltpu.VMEM((_NT, P), jnp.int32),
        pltpu.VMEM((_NT, P), jnp.float32),
        pltpu.VMEM((_NT, P), jnp.float32),
        pltpu.VMEM((_NT, P, C), jnp.float32),
        pltpu.VMEM((_NT, P, C), jnp.float32),
        pltpu.VMEM((C, P), jnp.float32),
        pltpu.SemaphoreType.DMA,
        pltpu.SemaphoreType.DMA,
    ]

    def body(xf, g3, p0, p1, p2, out, xv, idx0, idx1, w0, w1, rows0, rows1,
             outv, sem0, sem1):
        wid = lax.axis_index("s") * _NC + lax.axis_index("c")
        tile_base = wid * ppt
        lane = lax.iota(jnp.int32, _L)
        tabs = (g3,) * 8 + (p0,) * 4 + (p1,) * 4 + (p2,) * 4
        idxs = (idx0, idx1)
        ws = (w0, w1)
        rows = (rows0, rows1)
        sems = (sem0, sem1)

        def make_prep(idx_r, w_r):
            def prep_group(g, c):
                sl = pl.ds(g * _L, _L)
                p6 = (lane + g * _L) * 6
                gx = plsc.load_gather(xv, [p6])
                gy = plsc.load_gather(xv, [p6 + 1])
                gz = plsc.load_gather(xv, [p6 + 2])
                coords = (gx, gy, gz)
                ix = (gx + 1.0) * 0.5 * (Wg - 1)
                iy = (gy + 1.0) * 0.5 * (Hg - 1)
                iz = (gz + 1.0) * 0.5 * (D - 1)
                xi = jnp.clip(ix.astype(jnp.int32), 0, Wg - 1)
                yi = jnp.clip(iy.astype(jnp.int32), 0, Hg - 1)
                zi = jnp.clip(iz.astype(jnp.int32), 0, D - 1)
                fx = ix - xi.astype(jnp.float32)
                fy = iy - yi.astype(jnp.float32)
                fz = iz - zi.astype(jnp.float32)
                dx = jnp.minimum(xi + 1, Wg - 1) - xi
                dy = (jnp.minimum(yi + 1, Hg - 1) - yi) * Wg
                dz = (jnp.minimum(zi + 1, D - 1) - zi) * (Wg * Hg)
                base3 = (zi * Hg + yi) * Wg + xi
                idx_r[0, sl] = base3
                idx_r[1, sl] = base3 + dx
                idx_r[2, sl] = base3 + dy
                idx_r[3, sl] = base3 + dy + dx
                idx_r[4, sl] = base3 + dz
                idx_r[5, sl] = base3 + dz + dx
                idx_r[6, sl] = base3 + dz + dy
                idx_r[7, sl] = base3 + dz + dy + dx
                ox = 1.0 - fx
                oy = 1.0 - fy
                oz = 1.0 - fz
                w_r[0, sl] = ox * oy * oz
                w_r[1, sl] = fx * oy * oz
                w_r[2, sl] = ox * fy * oz
                w_r[3, sl] = fx * fy * oz
                w_r[4, sl] = ox * oy * fz
                w_r[5, sl] = fx * oy * fz
                w_r[6, sl] = ox * fy * fz
                w_r[7, sl] = fx * fy * fz
                for k, (a, b) in enumerate(_PLANE_DIMID):
                    PHk, PWk = pdims[k]
                    gu = coords[a]
                    gv = coords[b]
                    iu = (gu + 1.0) * 0.5 * (PWk - 1)
                    iv = (gv + 1.0) * 0.5 * (PHk - 1)
                    ui = jnp.clip(iu.astype(jnp.int32), 0, PWk - 1)
                    vi = jnp.clip(iv.astype(jnp.int32), 0, PHk - 1)
                    fu = iu - ui.astype(jnp.float32)
                    fv = iv - vi.astype(jnp.float32)
                    du = jnp.minimum(ui + 1, PWk - 1) - ui
                    dv = (jnp.minimum(vi + 1, PHk - 1) - vi) * PWk
                    r = vi * PWk + ui
                    t0 = 8 + 4 * k
                    idx_r[t0 + 0, sl] = r
                    idx_r[t0 + 1, sl] = r + du
                    idx_r[t0 + 2, sl] = r + dv
                    idx_r[t0 + 3, sl] = r + dv + du
                    ou = 1.0 - fu
                    ov = 1.0 - fv
                    w_r[t0 + 0, sl] = ou * ov
                    w_r[t0 + 1, sl] = fu * ov
                    w_r[t0 + 2, sl] = ou * fv
                    w_r[t0 + 3, sl] = fu * fv
                return c
            return prep_group

        def make_combine(rows_r, w_r):
            def combine_group(g, c):
                sl = pl.ds(g * _L, _L)
                pvec = lane + g * _L
                wvs = [w_r[t, sl] for t in range(_NT)]
                for f in range(C):
                    fvec = jnp.full((_L,), f, jnp.int32)
                    a = None
                    for t in range(8):
                        tv = jnp.full((_L,), t, jnp.int32)
                        v = plsc.load_gather(rows_r, [tv, pvec, fvec])
                        a = wvs[t] * v if a is None else a + wvs[t] * v
                    for k in range(3):
                        m = None
                        for t in range(8 + 4 * k, 12 + 4 * k):
                            tv = jnp.full((_L,), t, jnp.int32)
                            v = plsc.load_gather(rows_r, [tv, pvec, fvec])
                            m = wvs[t] * v if m is None else m + wvs[t] * v
                        a = a * m
                    outv[f, sl] = a
                return c
            return combine_group

        preps = (make_prep(idx0, w0), make_prep(idx1, w1))
        combines = (make_combine(rows0, w0), make_combine(rows1, w1))

        def stage_prep_fire(cc, par):
            base = tile_base + cc * P
            pltpu.sync_copy(xf.at[pl.ds(base * 6, P * 6)], xv)
            lax.fori_loop(0, ngroups, preps[par], 0)
            for t in range(_NT):
                pltpu.async_copy(tabs[t].at[idxs[par].at[t]], rows[par].at[t],
                                 sems[par])

        def wait_combine_store(cc, par):
            for t in range(_NT):
                pltpu.make_async_copy(tabs[t].at[idxs[par].at[t]],
                                      rows[par].at[t], sems[par]).wait()
            lax.fori_loop(0, ngroups, combines[par], 0)
            base = tile_base + cc * P
            pltpu.sync_copy(outv, out.at[:, pl.ds(base, P)])

        if pipelined:
            stage_prep_fire(0, 0)

            def pair(j, c):
                c0 = 2 * j
                stage_prep_fire(c0 + 1, 1)
                wait_combine_store(c0, 0)

                @pl.when(c0 + 2 < nchunks)
                def _fire_next():
                    stage_prep_fire(c0 + 2, 0)

                wait_combine_store(c0 + 1, 1)
                return c

            lax.fori_loop(0, nchunks // 2, pair, 0)
        else:
            def chunk(kk, c):
                stage_prep_fire(kk, 0)
                wait_combine_store(kk, 0)
                return c

            lax.fori_loop(0, nchunks, chunk, 0)

    return pl.kernel(body, out_type=out_type, mesh=mesh,
                     scratch_types=scratch, interpret=interpret,
                     compiler_params=pltpu.CompilerParams(
                         needs_layout_passes=False,
                         use_tc_tiling_on_sc=False))


def _make_relayout_kernel(C, sizes, interpret=False):
    U = 1024
    counts = [n // (U * _NW) for n in sizes]
    assert all(n % (U * _NW) == 0 for n in sizes)

    mesh = plsc.VectorSubcoreMesh(core_axis_name="c", subcore_axis_name="s",
                                  num_cores=_NC, num_subcores=_NS)
    out_type = tuple(jax.ShapeDtypeStruct((n, C), jnp.float32) for n in sizes)
    scratch = [
        pltpu.VMEM((C, U), jnp.float32),
        pltpu.VMEM((U, C), jnp.float32),
        pltpu.SemaphoreType.DMA,
    ]

    def body(*refs):
        ins = refs[:len(sizes)]
        outs = refs[len(sizes):2 * len(sizes)]
        inb, stg, sem = refs[2 * len(sizes):]
        wid = lax.axis_index("s") * _NC + lax.axis_index("c")
        lane16 = lax.iota(jnp.int32, _L)

        def do_table(t_in, t_out, nunits):
            ubase = wid * nunits

            def unit(i, c):
                off = (ubase + i) * U
                for ch in range(C):
                    pltpu.async_copy(t_in.at[ch, pl.ds(off, U)], inb.at[ch],
                                     sem)
                for ch in range(C):
                    pltpu.make_async_copy(t_in.at[ch, pl.ds(off, U)],
                                          inb.at[ch], sem).wait()

                def grp(i16, cc):
                    nidx = lane16 + i16 * _L
                    for ch in range(C):
                        v = inb[ch, pl.ds(i16 * _L, _L)]
                        plsc.store_scatter(stg, [nidx, jnp.full((_L,), ch,
                                                                jnp.int32)], v)
                    return cc

                lax.fori_loop(0, U // _L, grp, 0)
                pltpu.sync_copy(stg, t_out.at[pl.ds(off, U), :])
                return c

            lax.fori_loop(0, nunits, unit, 0)

        for t_in, t_out, cnt in zip(ins, outs, counts):
            do_table(t_in, t_out, cnt)

    return pl.kernel(body, out_type=out_type, mesh=mesh,
                     scratch_types=scratch, interpret=interpret,
                     compiler_params=pltpu.CompilerParams(
                         needs_layout_passes=False,
                         use_tc_tiling_on_sc=False))


def kernel(x, feature_grid_3d, plane0, plane1, plane2):
    B = x.shape[0]
    C = feature_grid_3d.shape[1]
    D, Hg, Wg = feature_grid_3d.shape[2:5]
    assert C == _L and B % (_NW * _L) == 0
    pdims = [(p.shape[2], p.shape[3]) for p in (plane0, plane1, plane2)]
    sizes = [D * Hg * Wg] + [ph * pw for ph, pw in pdims]
    rk = _make_relayout_kernel(C, tuple(sizes))
    g3t, p0t, p1t, p2t = rk(feature_grid_3d.reshape(C, -1),
                            plane0.reshape(C, -1),
                            plane1.reshape(C, -1),
                            plane2.reshape(C, -1))
    k = _make_sc_kernel(B, C, (D, Hg, Wg), tuple(pdims))
    return k(x.reshape(-1), g3t, p0t, p1t, p2t)
